# Initial kernel scaffold; baseline (speedup 1.0000x reference)
import jax
import jax.numpy as jnp
from jax.experimental import pallas as pl


def kernel(x, edge_index, W1a, b1a, W1b, b1b, g1, be1, W2a, b2a, W2b, b2b, g2, be2, Wf, bf):
    raise NotImplementedError("write your pallas kernel here")


# speedup vs baseline: 3.1529x; 3.1529x over previous
import functools

import jax
import jax.numpy as jnp
from jax import lax
from jax.experimental import pallas as pl
from jax.experimental.pallas import tpu as pltpu
from jax.experimental.pallas import tpu_sc as plsc

N = 10000
E = 320000
D = 128
OUT = 128
BN_EPS = 1e-5

NC = 2
NS = 16
NW = NC * NS
CHUNK = 128
CH_PER_W = 80
E_PAD = NW * CH_PER_W * CHUNK
N_PAD = 10240
ROWS_PER_TILE = N_PAD // NS


def _sc_aggregate(feat, srcp, dstp, zeros):
    mesh = plsc.VectorSubcoreMesh(core_axis_name="c", subcore_axis_name="s")

    @functools.partial(
        pl.kernel,
        mesh=mesh,
        out_type=jax.ShapeDtypeStruct((NC, N_PAD, D), jnp.float32),
        scratch_types=[
            pltpu.VMEM((CH_PER_W, CHUNK), jnp.int32),
            pltpu.VMEM((CH_PER_W, CHUNK), jnp.int32),
            pltpu.VMEM((CHUNK, D), jnp.float32),
            pltpu.VMEM_SHARED((N_PAD, D), jnp.float32),
            pltpu.SemaphoreType.DMA,
        ],
    )
    def k(feat_hbm, src_hbm, dst_hbm, z_hbm, out_hbm, sidx, didx, rows, acc, sem):
        cid = lax.axis_index("c")
        sid = lax.axis_index("s")
        wid = sid * NC + cid

        pltpu.sync_copy(z_hbm.at[pl.ds(sid * ROWS_PER_TILE, ROWS_PER_TILE)],
                        acc.at[pl.ds(sid * ROWS_PER_TILE, ROWS_PER_TILE)])
        plsc.subcore_barrier()

        pltpu.sync_copy(src_hbm.at[pl.ds(wid * CH_PER_W, CH_PER_W)], sidx)
        pltpu.sync_copy(dst_hbm.at[pl.ds(wid * CH_PER_W, CH_PER_W)], didx)

        @pl.loop(0, CH_PER_W)
        def _(j):
            pltpu.async_copy(feat_hbm.at[sidx.at[j]], rows, sem).wait()
            pltpu.sync_copy(rows, acc.at[didx.at[j]], add=True)

        plsc.subcore_barrier()
        pltpu.sync_copy(acc.at[pl.ds(sid * ROWS_PER_TILE, ROWS_PER_TILE)],
                        out_hbm.at[cid, pl.ds(sid * ROWS_PER_TILE, ROWS_PER_TILE)])

    return k(feat, srcp, dstp, zeros)


def _tc_layer1(x, p, W1a, b1a, W1b, b1b, g1, be1):
    def body(x_ref, p_ref, wa_ref, ba_ref, wb_ref, bb_ref, g_ref, be_ref, o_ref):
        agg = x_ref[...] + p_ref[0, :N, :] + p_ref[1, :N, :]
        t = jnp.dot(agg, wa_ref[...], preferred_element_type=jnp.float32)
        t = jnp.maximum(t + ba_ref[...], 0.0)
        h = jnp.dot(t, wb_ref[...], preferred_element_type=jnp.float32)
        h = h + bb_ref[...]
        mean = jnp.mean(h, axis=0, keepdims=True)
        var = jnp.mean((h - mean) ** 2, axis=0, keepdims=True)
        h = (h - mean) * lax.rsqrt(var + BN_EPS) * g_ref[...] + be_ref[...]
        o_ref[...] = jnp.maximum(h, 0.0)

    return pl.pallas_call(
        body,
        out_shape=jax.ShapeDtypeStruct((N, D), jnp.float32),
    )(x, p, W1a, b1a.reshape(1, D), W1b, b1b.reshape(1, D),
      g1.reshape(1, D), be1.reshape(1, D))


def _tc_layer2(h1, q, W2a, b2a, W2b, b2b, g2, be2, Wf, bf):
    def body(x_ref, p_ref, wa_ref, ba_ref, wb_ref, bb_ref, g_ref, be_ref,
             wf_ref, bf_ref, o_ref):
        agg = x_ref[...] + p_ref[0, :N, :] + p_ref[1, :N, :]
        t = jnp.dot(agg, wa_ref[...], preferred_element_type=jnp.float32)
        t = jnp.maximum(t + ba_ref[...], 0.0)
        h = jnp.dot(t, wb_ref[...], preferred_element_type=jnp.float32)
        h = h + bb_ref[...]
        mean = jnp.mean(h, axis=0, keepdims=True)
        var = jnp.mean((h - mean) ** 2, axis=0, keepdims=True)
        h = (h - mean) * lax.rsqrt(var + BN_EPS) * g_ref[...] + be_ref[...]
        o_ref[...] = jnp.dot(h, wf_ref[...],
                             preferred_element_type=jnp.float32) + bf_ref[...]

    return pl.pallas_call(
        body,
        out_shape=jax.ShapeDtypeStruct((N, OUT), jnp.float32),
    )(h1, q, W2a, b2a.reshape(1, D), W2b, b2b.reshape(1, D),
      g2.reshape(1, D), be2.reshape(1, D), Wf, bf.reshape(1, OUT))


def kernel(x, edge_index, W1a, b1a, W1b, b1b, g1, be1,
           W2a, b2a, W2b, b2b, g2, be2, Wf, bf):
    src = edge_index[0].astype(jnp.int32)
    dst = edge_index[1].astype(jnp.int32)
    npad = E_PAD - E
    srcp = jnp.concatenate([src, jnp.zeros((npad,), jnp.int32)])
    dstp = jnp.concatenate([dst, jnp.full((npad,), N, jnp.int32)])
    srcp = srcp.reshape(NW * CH_PER_W, CHUNK)
    dstp = dstp.reshape(NW * CH_PER_W, CHUNK)
    zeros = jnp.zeros((N_PAD, D), jnp.float32)

    p = _sc_aggregate(x, srcp, dstp, zeros)
    h1 = _tc_layer1(x, p, W1a, b1a, W1b, b1b, g1, be1)
    q = _sc_aggregate(h1, srcp, dstp, zeros)
    return _tc_layer2(h1, q, W2a, b2a, W2b, b2b, g2, be2, Wf, bf)


# speedup vs baseline: 3.1561x; 1.0010x over previous
import functools

import jax
import jax.numpy as jnp
from jax import lax
from jax.experimental import pallas as pl
from jax.experimental.pallas import tpu as pltpu
from jax.experimental.pallas import tpu_sc as plsc

N = 10000
E = 320000
D = 128
OUT = 128
BN_EPS = 1e-5

NC = 2
NS = 16
NW = NC * NS
CHUNK = 128
CH_PER_W = 80
E_PAD = NW * CH_PER_W * CHUNK
N_PAD = 10240
ROWS_PER_TILE = N_PAD // NS


def _sc_aggregate(feat, srcp, dstp, zeros):
    mesh = plsc.VectorSubcoreMesh(core_axis_name="c", subcore_axis_name="s")

    @functools.partial(
        pl.kernel,
        mesh=mesh,
        out_type=jax.ShapeDtypeStruct((NC, N_PAD, D), jnp.float32),
        scratch_types=[
            pltpu.VMEM((CH_PER_W, CHUNK), jnp.int32),
            pltpu.VMEM((CH_PER_W, CHUNK), jnp.int32),
            pltpu.VMEM((CHUNK, D), jnp.float32),
            pltpu.VMEM_SHARED((N_PAD, D), jnp.float32),
            pltpu.SemaphoreType.DMA,
        ],
    )
    def k(feat_hbm, src_hbm, dst_hbm, z_hbm, out_hbm, sidx, didx,
          rows, acc, sem):
        cid = lax.axis_index("c")
        sid = lax.axis_index("s")
        wid = sid * NC + cid

        pltpu.sync_copy(z_hbm.at[pl.ds(sid * ROWS_PER_TILE, ROWS_PER_TILE)],
                        acc.at[pl.ds(sid * ROWS_PER_TILE, ROWS_PER_TILE)])
        plsc.subcore_barrier()

        pltpu.sync_copy(src_hbm.at[pl.ds(wid * CH_PER_W, CH_PER_W)], sidx)
        pltpu.sync_copy(dst_hbm.at[pl.ds(wid * CH_PER_W, CH_PER_W)], didx)

        @pl.loop(0, CH_PER_W)
        def _(j):
            pltpu.async_copy(feat_hbm.at[sidx.at[j]], rows, sem).wait()
            pltpu.sync_copy(rows, acc.at[didx.at[j]], add=True)

        plsc.subcore_barrier()
        pltpu.sync_copy(acc.at[pl.ds(sid * ROWS_PER_TILE, ROWS_PER_TILE)],
                        out_hbm.at[cid, pl.ds(sid * ROWS_PER_TILE, ROWS_PER_TILE)])

    return k(feat, srcp, dstp, zeros)


def _tc_layer1(x, p, W1a, b1a, W1b, b1b, g1, be1):
    def body(x_ref, p_ref, wa_ref, ba_ref, wb_ref, bb_ref, g_ref, be_ref, o_ref):
        agg = x_ref[...] + p_ref[0, :N, :] + p_ref[1, :N, :]
        t = jnp.dot(agg, wa_ref[...], preferred_element_type=jnp.float32)
        t = jnp.maximum(t + ba_ref[...], 0.0)
        h = jnp.dot(t, wb_ref[...], preferred_element_type=jnp.float32)
        h = h + bb_ref[...]
        mean = jnp.mean(h, axis=0, keepdims=True)
        var = jnp.mean((h - mean) ** 2, axis=0, keepdims=True)
        h = (h - mean) * lax.rsqrt(var + BN_EPS) * g_ref[...] + be_ref[...]
        o_ref[...] = jnp.maximum(h, 0.0)

    return pl.pallas_call(
        body,
        out_shape=jax.ShapeDtypeStruct((N, D), jnp.float32),
    )(x, p, W1a, b1a.reshape(1, D), W1b, b1b.reshape(1, D),
      g1.reshape(1, D), be1.reshape(1, D))


def _tc_layer2(h1, q, W2a, b2a, W2b, b2b, g2, be2, Wf, bf):
    def body(x_ref, p_ref, wa_ref, ba_ref, wb_ref, bb_ref, g_ref, be_ref,
             wf_ref, bf_ref, o_ref):
        agg = x_ref[...] + p_ref[0, :N, :] + p_ref[1, :N, :]
        t = jnp.dot(agg, wa_ref[...], preferred_element_type=jnp.float32)
        t = jnp.maximum(t + ba_ref[...], 0.0)
        h = jnp.dot(t, wb_ref[...], preferred_element_type=jnp.float32)
        h = h + bb_ref[...]
        mean = jnp.mean(h, axis=0, keepdims=True)
        var = jnp.mean((h - mean) ** 2, axis=0, keepdims=True)
        h = (h - mean) * lax.rsqrt(var + BN_EPS) * g_ref[...] + be_ref[...]
        o_ref[...] = jnp.dot(h, wf_ref[...],
                             preferred_element_type=jnp.float32) + bf_ref[...]

    return pl.pallas_call(
        body,
        out_shape=jax.ShapeDtypeStruct((N, OUT), jnp.float32),
    )(h1, q, W2a, b2a.reshape(1, D), W2b, b2b.reshape(1, D),
      g2.reshape(1, D), be2.reshape(1, D), Wf, bf.reshape(1, OUT))


def kernel(x, edge_index, W1a, b1a, W1b, b1b, g1, be1,
           W2a, b2a, W2b, b2b, g2, be2, Wf, bf):
    src = edge_index[0].astype(jnp.int32)
    dst = edge_index[1].astype(jnp.int32)
    npad = E_PAD - E
    srcp = jnp.concatenate([src, jnp.zeros((npad,), jnp.int32)])
    pad_dst = N + (jnp.arange(npad, dtype=jnp.int32) % (N_PAD - N))
    dstp = jnp.concatenate([dst, pad_dst])
    srcp = srcp.reshape(NW * CH_PER_W, CHUNK)
    dstp = dstp.reshape(NW * CH_PER_W, CHUNK)
    zeros = jnp.zeros((N_PAD, D), jnp.float32)

    p = _sc_aggregate(x, srcp, dstp, zeros)
    h1 = _tc_layer1(x, p, W1a, b1a, W1b, b1b, g1, be1)
    q = _sc_aggregate(h1, srcp, dstp, zeros)
    return _tc_layer2(h1, q, W2a, b2a, W2b, b2b, g2, be2, Wf, bf)
